# baseline (speedup 1.0000x reference)
import jax
import jax.numpy as jnp
from jax import lax
from jax.experimental import pallas as pl
from jax.experimental.pallas import tpu as pltpu
from jax.experimental.pallas import tpu_sc as plsc

_VOCAB, _D, _H, _OUT = 1000000, 64, 512, 3
_B, _L = 4096, 200

_NC, _NS, _LANES = 2, 16, 16
_NW = _NC * _NS
_SEQ_PER_W = _B // _NW
_C0, _C1 = 128, _L - 128


def _pool_body(vec_hbm, tab_hbm, e0_hbm, out_hbm, idx_v, rows_a, rows_b,
               emb0_v, out_v, sem):
    cid = lax.axis_index("c")
    sid = lax.axis_index("s")
    wid = sid * _NC + cid
    base = wid * _SEQ_PER_W

    pltpu.sync_copy(e0_hbm, emb0_v)

    idx_v[pl.ds(192, _LANES)] = jnp.ones((_LANES,), jnp.int32)

    zero_f = jnp.zeros((_LANES,), jnp.float32)

    def seq_body(i, carry):
        b = base + i
        pltpu.sync_copy(vec_hbm.at[pl.ds(b * _L, _L)], idx_v.at[pl.ds(0, _L)])
        cp0 = pltpu.async_copy(tab_hbm.at[idx_v.at[pl.ds(0, _C0)]], rows_a, sem)
        cp1 = pltpu.async_copy(tab_hbm.at[idx_v.at[pl.ds(_C0, _C1)]], rows_b, sem)
        cp0.wait()
        cp1.wait()

        n0v = jnp.zeros((_LANES,), jnp.int32)
        for j in range(13):
            v = idx_v[pl.ds(j * _LANES, _LANES)]
            n0v = n0v + jnp.where(v == 0, 1, 0).astype(jnp.int32)
        n0 = n0v[0]
        for k in range(1, _LANES):
            n0 = n0 + n0v[k]

        def make_row_body(rows_ref):
            def row_body(t, accs_in):
                a0, a1, a2, a3 = accs_in
                a0 = a0 + rows_ref[t, pl.ds(0, _LANES)]
                a1 = a1 + rows_ref[t, pl.ds(_LANES, _LANES)]
                a2 = a2 + rows_ref[t, pl.ds(2 * _LANES, _LANES)]
                a3 = a3 + rows_ref[t, pl.ds(3 * _LANES, _LANES)]
                return (a0, a1, a2, a3)
            return row_body

        accs = lax.fori_loop(0, _C0, make_row_body(rows_a),
                             (zero_f, zero_f, zero_f, zero_f))
        accs = lax.fori_loop(0, _C1, make_row_body(rows_b), accs)

        n0f = jnp.broadcast_to(n0.astype(jnp.float32), (_LANES,))
        inv = 1.0 / ((jnp.float32(_L) - n0f) + 1e-8)
        for j in range(4):
            e = emb0_v[pl.ds(j * _LANES, _LANES)]
            out_v[pl.ds(i * _D + j * _LANES, _LANES)] = (accs[j] - n0f * e) * inv
        return carry

    lax.fori_loop(0, _SEQ_PER_W, seq_body, 0)
    pltpu.sync_copy(out_v, out_hbm.at[pl.ds(base * _D, _SEQ_PER_W * _D)])


@jax.jit
def _pool(vec_flat, emb_table, e0):
    mesh = plsc.VectorSubcoreMesh(core_axis_name="c", subcore_axis_name="s")
    out_flat = pl.kernel(
        _pool_body,
        out_type=jax.ShapeDtypeStruct((_B * _D,), jnp.float32),
        mesh=mesh,
        scratch_types=[
            pltpu.VMEM((208,), jnp.int32),
            pltpu.VMEM((_C0, _D), jnp.float32),
            pltpu.VMEM((_C1, _D), jnp.float32),
            pltpu.VMEM((_D,), jnp.float32),
            pltpu.VMEM((_SEQ_PER_W * _D,), jnp.float32),
            pltpu.SemaphoreType.DMA,
        ],
        compiler_params=pltpu.CompilerParams(use_tc_tiling_on_sc=False),
    )(vec_flat, emb_table, e0)
    return out_flat.reshape(_B, _D)


_BLK = 512


def _mlp_body(x_ref, w1_ref, b1_ref, g1_ref, be1_ref, w2_ref, b2_ref, g2_ref,
              be2_ref, w3_ref, b3_ref, o_ref):
    def ln(h, g, be):
        mu = jnp.mean(h, axis=-1, keepdims=True)
        var = jnp.mean(jnp.square(h - mu), axis=-1, keepdims=True)
        return (h - mu) * lax.rsqrt(var + 1e-5) * g + be

    x = x_ref[...]
    h = jnp.dot(x, w1_ref[...], preferred_element_type=jnp.float32) + b1_ref[...]
    h = jnp.maximum(ln(h, g1_ref[...], be1_ref[...]), 0.0)
    h = jnp.dot(h, w2_ref[...], preferred_element_type=jnp.float32) + b2_ref[...]
    h = jnp.maximum(ln(h, g2_ref[...], be2_ref[...]), 0.0)
    o_ref[...] = jnp.dot(h, w3_ref[...], preferred_element_type=jnp.float32) + b3_ref[...]


@jax.jit
def _mlp(x, W1, b1, g1, be1, W2, b2, g2, be2, W3p, b3p):
    grid = (_B // _BLK,)
    full = lambda shape: pl.BlockSpec(shape, lambda i: (0, 0))
    return pl.pallas_call(
        _mlp_body,
        grid=grid,
        in_specs=[
            pl.BlockSpec((_BLK, _D), lambda i: (i, 0)),
            full((_D, _H)), full((1, _H)), full((1, _H)), full((1, _H)),
            full((_H, _H)), full((1, _H)), full((1, _H)), full((1, _H)),
            full((_H, 128)), full((1, 128)),
        ],
        out_specs=pl.BlockSpec((_BLK, 128), lambda i: (i, 0)),
        out_shape=jax.ShapeDtypeStruct((_B, 128), jnp.float32),
    )(x, W1, b1, g1, be1, W2, b2, g2, be2, W3p, b3p)


def kernel(vector, emb_table, W1, b1, g1, be1, W2, b2, g2, be2, W3, b3):
    vec_flat = vector.astype(jnp.int32).reshape(-1)
    pooled = _pool(vec_flat, emb_table, emb_table[0])
    W3p = jnp.pad(W3, ((0, 0), (0, 128 - _OUT)))
    b3p = jnp.pad(b3, (0, 128 - _OUT))
    out = _mlp(
        pooled,
        W1, b1.reshape(1, _H), g1.reshape(1, _H), be1.reshape(1, _H),
        W2, b2.reshape(1, _H), g2.reshape(1, _H), be2.reshape(1, _H),
        W3p, b3p.reshape(1, 128),
    )
    return out[:, :_OUT]


# speedup vs baseline: 1.2529x; 1.2529x over previous
import jax
import jax.numpy as jnp
from jax import lax
from jax.experimental import pallas as pl
from jax.experimental.pallas import tpu as pltpu
from jax.experimental.pallas import tpu_sc as plsc

_VOCAB, _D, _H, _OUT = 1000000, 64, 512, 3
_B, _L = 4096, 200

_NC, _NS, _LANES = 2, 16, 16
_NW = _NC * _NS
_SEQ_PER_W = _B // _NW
_C0, _C1 = 128, _L - 128


def _pool_body(vec_hbm, tab_hbm, e0_hbm, out_hbm, idx_v, ra0, rb0, ra1, rb1,
               emb0_v, out_v, sem0, sem1):
    cid = lax.axis_index("c")
    sid = lax.axis_index("s")
    wid = sid * _NC + cid
    base = wid * _SEQ_PER_W

    pltpu.sync_copy(e0_hbm, emb0_v)
    pltpu.sync_copy(vec_hbm.at[pl.ds(base * _L, _SEQ_PER_W * _L)], idx_v)

    lane = lax.iota(jnp.int32, _LANES)
    zero_f = jnp.zeros((_LANES,), jnp.float32)

    def issue(i, rows_a, rows_b, sem):
        off = i * _L
        pltpu.async_copy(tab_hbm.at[idx_v.at[pl.ds(off, _C0)]], rows_a, sem)
        pltpu.async_copy(tab_hbm.at[idx_v.at[pl.ds(off + _C0, _C1)]], rows_b, sem)

    def drain(rows_a, rows_b, sem):
        pltpu.make_async_copy(tab_hbm.at[idx_v.at[pl.ds(0, _C0)]], rows_a, sem).wait()
        pltpu.make_async_copy(tab_hbm.at[idx_v.at[pl.ds(0, _C1)]], rows_b, sem).wait()

    def process(i, rows_a, rows_b):
        off = i * _L
        n0v = jnp.zeros((_LANES,), jnp.int32)
        for j in range(12):
            v = idx_v[pl.ds(off + j * _LANES, _LANES)]
            n0v = n0v + jnp.where(v == 0, 1, 0).astype(jnp.int32)
        w = idx_v[pl.ds(off + _L - _LANES, _LANES)]
        n0v = n0v + jnp.where((w == 0) & (lane >= 8), 1, 0).astype(jnp.int32)
        n0 = n0v[0]
        for k in range(1, _LANES):
            n0 = n0 + n0v[k]

        def make_row_body(rows_ref):
            def row_body(t, accs_in):
                a0, a1, a2, a3 = accs_in
                a0 = a0 + rows_ref[t, pl.ds(0, _LANES)]
                a1 = a1 + rows_ref[t, pl.ds(_LANES, _LANES)]
                a2 = a2 + rows_ref[t, pl.ds(2 * _LANES, _LANES)]
                a3 = a3 + rows_ref[t, pl.ds(3 * _LANES, _LANES)]
                return (a0, a1, a2, a3)
            return row_body

        accs = lax.fori_loop(0, _C0, make_row_body(rows_a),
                             (zero_f, zero_f, zero_f, zero_f), unroll=8)
        accs = lax.fori_loop(0, _C1, make_row_body(rows_b), accs, unroll=8)

        n0f = jnp.broadcast_to(n0.astype(jnp.float32), (_LANES,))
        inv = 1.0 / ((jnp.float32(_L) - n0f) + 1e-8)
        for j in range(4):
            e = emb0_v[pl.ds(j * _LANES, _LANES)]
            out_v[pl.ds(i * _D + j * _LANES, _LANES)] = (accs[j] - n0f * e) * inv

    issue(0, ra0, rb0, sem0)

    def outer(k, carry):
        i2 = 2 * k
        issue(i2 + 1, ra1, rb1, sem1)
        drain(ra0, rb0, sem0)
        process(i2, ra0, rb0)
        issue(jnp.minimum(i2 + 2, _SEQ_PER_W - 1), ra0, rb0, sem0)
        drain(ra1, rb1, sem1)
        process(i2 + 1, ra1, rb1)
        return carry

    lax.fori_loop(0, _SEQ_PER_W // 2, outer, 0)
    drain(ra0, rb0, sem0)

    pltpu.sync_copy(out_v, out_hbm.at[pl.ds(base * _D, _SEQ_PER_W * _D)])


@jax.jit
def _pool(vec_flat, emb_table, e0):
    mesh = plsc.VectorSubcoreMesh(core_axis_name="c", subcore_axis_name="s")
    out_flat = pl.kernel(
        _pool_body,
        out_type=jax.ShapeDtypeStruct((_B * _D,), jnp.float32),
        mesh=mesh,
        scratch_types=[
            pltpu.VMEM((_SEQ_PER_W * _L,), jnp.int32),
            pltpu.VMEM((_C0, _D), jnp.float32),
            pltpu.VMEM((_C1, _D), jnp.float32),
            pltpu.VMEM((_C0, _D), jnp.float32),
            pltpu.VMEM((_C1, _D), jnp.float32),
            pltpu.VMEM((_D,), jnp.float32),
            pltpu.VMEM((_SEQ_PER_W * _D,), jnp.float32),
            pltpu.SemaphoreType.DMA,
            pltpu.SemaphoreType.DMA,
        ],
        compiler_params=pltpu.CompilerParams(use_tc_tiling_on_sc=False),
    )(vec_flat, emb_table, e0)
    return out_flat.reshape(_B, _D)


_BLK = 512


def _mlp_body(x_ref, w1_ref, b1_ref, g1_ref, be1_ref, w2_ref, b2_ref, g2_ref,
              be2_ref, w3_ref, b3_ref, o_ref):
    def ln(h, g, be):
        mu = jnp.mean(h, axis=-1, keepdims=True)
        var = jnp.mean(jnp.square(h - mu), axis=-1, keepdims=True)
        return (h - mu) * lax.rsqrt(var + 1e-5) * g + be

    x = x_ref[...]
    h = jnp.dot(x, w1_ref[...], preferred_element_type=jnp.float32) + b1_ref[...]
    h = jnp.maximum(ln(h, g1_ref[...], be1_ref[...]), 0.0)
    h = jnp.dot(h, w2_ref[...], preferred_element_type=jnp.float32) + b2_ref[...]
    h = jnp.maximum(ln(h, g2_ref[...], be2_ref[...]), 0.0)
    o_ref[...] = jnp.dot(h, w3_ref[...], preferred_element_type=jnp.float32) + b3_ref[...]


@jax.jit
def _mlp(x, W1, b1, g1, be1, W2, b2, g2, be2, W3p, b3p):
    grid = (_B // _BLK,)
    full = lambda shape: pl.BlockSpec(shape, lambda i: (0, 0))
    return pl.pallas_call(
        _mlp_body,
        grid=grid,
        in_specs=[
            pl.BlockSpec((_BLK, _D), lambda i: (i, 0)),
            full((_D, _H)), full((1, _H)), full((1, _H)), full((1, _H)),
            full((_H, _H)), full((1, _H)), full((1, _H)), full((1, _H)),
            full((_H, 128)), full((1, 128)),
        ],
        out_specs=pl.BlockSpec((_BLK, 128), lambda i: (i, 0)),
        out_shape=jax.ShapeDtypeStruct((_B, 128), jnp.float32),
    )(x, W1, b1, g1, be1, W2, b2, g2, be2, W3p, b3p)


def kernel(vector, emb_table, W1, b1, g1, be1, W2, b2, g2, be2, W3, b3):
    vec_flat = vector.astype(jnp.int32).reshape(-1)
    pooled = _pool(vec_flat, emb_table, emb_table[0])
    W3p = jnp.pad(W3, ((0, 0), (0, 128 - _OUT)))
    b3p = jnp.pad(b3, (0, 128 - _OUT))
    out = _mlp(
        pooled,
        W1, b1.reshape(1, _H), g1.reshape(1, _H), be1.reshape(1, _H),
        W2, b2.reshape(1, _H), g2.reshape(1, _H), be2.reshape(1, _H),
        W3p, b3p.reshape(1, 128),
    )
    return out[:, :_OUT]
